# baseline (speedup 1.0000x reference)
import functools

import jax
import jax.numpy as jnp
from jax import lax
from jax.experimental import pallas as pl
from jax.experimental.pallas import tpu as pltpu
from jax.experimental.pallas import tpu_sc as plsc

BATCH = 128
DIM = 2048
NBLK = DIM // 128
NUM_WORKERS = 32
B_PER_W = BATCH // NUM_WORKERS


def _sc_gather(t_pad, nw_tab, iw_tab):
    mesh = plsc.VectorSubcoreMesh(core_axis_name="c", subcore_axis_name="s")

    @functools.partial(
        pl.kernel,
        mesh=mesh,
        out_type=(
            jax.ShapeDtypeStruct((BATCH, DIM), jnp.float32),
            jax.ShapeDtypeStruct((BATCH, DIM), jnp.float32),
        ),
        scratch_types=[
            pltpu.VMEM((8,), jnp.int32),
            pltpu.VMEM((B_PER_W, DIM), jnp.float32),
            pltpu.VMEM((B_PER_W, DIM), jnp.float32),
            pltpu.SemaphoreType.DMA,
            pltpu.SemaphoreType.DMA,
        ],
    )
    def k(t_hbm, nw_hbm, iw_hbm, nw_out, iw_out, idx_v, nw_v, iw_v,
          sem_g, sem_o):
        wid = lax.axis_index("s") * 2 + lax.axis_index("c")
        base = wid * B_PER_W

        pltpu.sync_copy(t_hbm.at[wid], idx_v)
        idx4 = idx_v.at[pl.ds(0, B_PER_W)]
        g_nw = pltpu.async_copy(nw_hbm.at[idx4], nw_v, sem_g)
        g_iw = pltpu.async_copy(iw_hbm.at[idx4], iw_v, sem_g)
        g_nw.wait()
        o_nw = pltpu.async_copy(nw_v, nw_out.at[pl.ds(base, B_PER_W)], sem_o)
        g_iw.wait()
        o_iw = pltpu.async_copy(iw_v, iw_out.at[pl.ds(base, B_PER_W)], sem_o)
        o_nw.wait()
        o_iw.wait()

    return k(t_pad, nw_tab, iw_tab)


def _tc_blend(x0_b, nz_b, nw_rows, iw_rows):
    def body(x0_ref, nz_ref, nw_ref, iw_ref, out_ref):
        nw = nw_ref[...][:, :, None, :]
        iw = iw_ref[...][:, :, None, :]
        out_ref[...] = iw * x0_ref[...] + nw * nz_ref[...]

    return pl.pallas_call(
        body,
        out_shape=jax.ShapeDtypeStruct((BATCH, NBLK, 2, 128), jnp.float32),
    )(x0_b, nz_b, nw_rows, iw_rows)


def kernel(x_0, t, task_id, noise, noise_weights, info_weights):
    del task_id
    t_pad = jnp.pad(t.astype(jnp.int32).reshape(NUM_WORKERS, B_PER_W),
                    ((0, 0), (0, 8 - B_PER_W)))
    x0_b = x_0.reshape(BATCH, NBLK, 128, 2).transpose(0, 1, 3, 2)
    nz_b = noise.reshape(BATCH, NBLK, 128, 2).transpose(0, 1, 3, 2)
    nw_rows, iw_rows = _sc_gather(t_pad, noise_weights, info_weights)
    out = _tc_blend(x0_b, nz_b,
                    nw_rows.reshape(BATCH, NBLK, 128),
                    iw_rows.reshape(BATCH, NBLK, 128))
    return out.transpose(0, 1, 3, 2).reshape(BATCH, DIM, 2)
